# Initial kernel scaffold; baseline (speedup 1.0000x reference)
import jax
import jax.numpy as jnp
from jax.experimental import pallas as pl


def kernel(logits, Qhat):
    raise NotImplementedError("write your pallas kernel here")


# speedup vs baseline: 154.5484x; 154.5484x over previous
import jax
import jax.numpy as jnp
from jax import lax
from jax.experimental import pallas as pl
from jax.experimental.pallas import tpu as pltpu

_ROWS = 32


def _aps_body(qhat_ref, x_ref, mask_ref):
    qhat = qhat_ref[0]
    x = x_ref[...]
    R, N = x.shape

    xmax = jnp.max(x, axis=1, keepdims=True)
    e = jnp.exp(x - xmax)
    z = jnp.sum(e, axis=1, keepdims=True)
    s = e / z
    total = jnp.sum(s, axis=1, keepdims=True)

    def val_step(_, carry):
        lo, hi = carry
        mid = (lo + hi) >> 1
        t = lax.bitcast_convert_type(mid, jnp.float32)
        ssum = jnp.sum(jnp.where(s >= t, s, 0.0), axis=1, keepdims=True)
        pred = ssum > qhat
        return jnp.where(pred, mid, lo), jnp.where(pred, hi, mid)

    lo0 = jnp.zeros((R, 1), jnp.int32)
    hi0 = jnp.full((R, 1), 0x3F800001, jnp.int32)
    lo, _ = lax.fori_loop(0, 31, val_step, (lo0, hi0))
    vstar = lax.bitcast_convert_type(lo, jnp.float32)

    gt = s > vstar
    eq = s == vstar
    s_gt = jnp.sum(jnp.where(gt, s, 0.0), axis=1, keepdims=True)
    m_eq = jnp.sum(eq.astype(jnp.int32), axis=1, keepdims=True)

    vsafe = jnp.where(vstar > 0.0, vstar, 1.0)
    r = jnp.floor((qhat - s_gt) / vsafe).astype(jnp.int32) + 1
    r = jnp.clip(r, 1, jnp.maximum(m_eq, 1))

    idx = lax.broadcasted_iota(jnp.int32, (R, N), 1)

    def tie_idx_general():
        def idx_step(_, carry):
            lo_j, hi_j = carry
            mid_j = (lo_j + hi_j) >> 1
            cnt = jnp.sum((eq & (idx <= mid_j)).astype(jnp.int32),
                          axis=1, keepdims=True)
            ge = cnt >= r
            return jnp.where(ge, lo_j, mid_j), jnp.where(ge, mid_j, hi_j)

        lo_j = jnp.full((R, 1), -1, jnp.int32)
        hi_j = jnp.full((R, 1), N - 1, jnp.int32)
        _, hi_j = lax.fori_loop(0, 17, idx_step, (lo_j, hi_j))
        return hi_j

    def tie_idx_single():
        return jnp.min(jnp.where(eq, idx, N), axis=1, keepdims=True)

    jstar = lax.cond(jnp.any(m_eq > 1), tie_idx_general, tie_idx_single)

    full_row = (total <= qhat) | (qhat >= 1.0)
    mask = (gt | (eq & (idx <= jstar))) | full_row
    mask_ref[...] = mask.astype(jnp.uint8)


def kernel(logits, Qhat):
    b, n = logits.shape
    qhat1 = jnp.asarray(Qhat, jnp.float32).reshape(1)
    mask_u8 = pl.pallas_call(
        _aps_body,
        grid=(b // _ROWS,),
        in_specs=[
            pl.BlockSpec(memory_space=pltpu.SMEM),
            pl.BlockSpec((_ROWS, n), lambda i: (i, 0)),
        ],
        out_specs=pl.BlockSpec((_ROWS, n), lambda i: (i, 0)),
        out_shape=jax.ShapeDtypeStruct((b, n), jnp.uint8),
    )(qhat1, logits)
    return (logits, mask_u8.astype(jnp.bool_))


# speedup vs baseline: 162.4382x; 1.0511x over previous
import jax
import jax.numpy as jnp
from jax import lax
from jax.experimental import pallas as pl
from jax.experimental.pallas import tpu as pltpu

_ROWS = 32


def _aps_body(qhat_ref, x_ref, mask_ref):
    qhat = qhat_ref[0]
    x = x_ref[...]
    R, N = x.shape

    xmax = jnp.max(x, axis=1, keepdims=True)
    e = jnp.exp(x - xmax)
    z = jnp.sum(e, axis=1, keepdims=True)
    s = e / z
    total = jnp.sum(s, axis=1, keepdims=True)

    rmax = jnp.float32(1.0) / z
    hi0 = lax.bitcast_convert_type(rmax, jnp.int32) + 1
    lo_v = jnp.maximum((total - qhat) / jnp.float32(2 * N), 0.0)
    lo0 = lax.bitcast_convert_type(lo_v, jnp.int32)

    def val_cond(carry):
        lo, hi = carry
        return jnp.any(hi - lo > 1)

    def val_step(carry):
        lo, hi = carry
        mid = (lo + hi) >> 1
        t = lax.bitcast_convert_type(mid, jnp.float32)
        ssum = jnp.sum(jnp.where(s >= t, s, 0.0), axis=1, keepdims=True)
        pred = ssum > qhat
        return jnp.where(pred, mid, lo), jnp.where(pred, hi, mid)

    lo, _ = lax.while_loop(val_cond, val_step, (lo0, hi0))
    vstar = lax.bitcast_convert_type(lo, jnp.float32)

    gt = s > vstar
    eq = s == vstar
    s_gt = jnp.sum(jnp.where(gt, s, 0.0), axis=1, keepdims=True)
    m_eq = jnp.sum(eq.astype(jnp.int32), axis=1, keepdims=True)

    vsafe = jnp.where(vstar > 0.0, vstar, 1.0)
    r = jnp.floor((qhat - s_gt) / vsafe).astype(jnp.int32) + 1
    r = jnp.clip(r, 1, jnp.maximum(m_eq, 1))

    idx = lax.broadcasted_iota(jnp.int32, (R, N), 1)

    def tie_idx_general():
        def idx_step(_, carry):
            lo_j, hi_j = carry
            mid_j = (lo_j + hi_j) >> 1
            cnt = jnp.sum((eq & (idx <= mid_j)).astype(jnp.int32),
                          axis=1, keepdims=True)
            ge = cnt >= r
            return jnp.where(ge, lo_j, mid_j), jnp.where(ge, mid_j, hi_j)

        lo_j = jnp.full((R, 1), -1, jnp.int32)
        hi_j = jnp.full((R, 1), N - 1, jnp.int32)
        _, hi_j = lax.fori_loop(0, 17, idx_step, (lo_j, hi_j))
        return hi_j

    def tie_idx_single():
        return jnp.min(jnp.where(eq, idx, N), axis=1, keepdims=True)

    jstar = lax.cond(jnp.any(m_eq > 1), tie_idx_general, tie_idx_single)

    full_row = (lo == 0) | (qhat >= 1.0)
    mask = (gt | (eq & (idx <= jstar))) | full_row
    mask_ref[...] = mask.astype(jnp.uint8)


def kernel(logits, Qhat):
    b, n = logits.shape
    qhat1 = jnp.asarray(Qhat, jnp.float32).reshape(1)
    mask_u8 = pl.pallas_call(
        _aps_body,
        grid=(b // _ROWS,),
        in_specs=[
            pl.BlockSpec(memory_space=pltpu.SMEM),
            pl.BlockSpec((_ROWS, n), lambda i: (i, 0)),
        ],
        out_specs=pl.BlockSpec((_ROWS, n), lambda i: (i, 0)),
        out_shape=jax.ShapeDtypeStruct((b, n), jnp.uint8),
    )(qhat1, logits)
    return (logits, mask_u8.astype(jnp.bool_))
